# baseline (device time: 112562 ns/iter reference)
import functools

import jax
import jax.numpy as jnp
from jax import lax
from jax.experimental import pallas as pl
from jax.experimental.pallas import tpu as pltpu

N_DEV = 8
SQ = 2048
D_MODEL = 1024
HEADS = 8
DH = 128
WIN = 128
QBLK = 256
KBLK = 512
CHUNK = SQ // N_DEV
SCALE = 0.08838834764831843


def kernel(x, Wq, K_ext, V_ext, Wo):
    xb = x[0]
    kb = K_ext[0]
    vb = V_ext[0]

    def body(x_ref, wq_ref, k_ref, v_ref, wo_ref, out_ref,
             ctx_ref, partial_ref, comm_ref, ag_ref,
             wqb_ref, wob_ref, kv_ref, send_sems, recv_sems, kv_sems):
        my = lax.axis_index("i")
        left = (my - 1) % N_DEV
        right = (my + 1) % N_DEV

        barrier_sem = pltpu.get_barrier_semaphore()
        for nbr in (left, right):
            pl.semaphore_signal(
                barrier_sem, inc=1,
                device_id=(nbr,), device_id_type=pl.DeviceIdType.MESH,
            )
        wqb_ref[:, :] = (wq_ref[:, :] * SCALE).astype(jnp.bfloat16)
        wob_ref[:, :] = wo_ref[:, :].astype(jnp.bfloat16)
        pl.semaphore_wait(barrier_sem, 2)

        def window_start(j):
            qb = (my - j) % N_DEV
            q0 = pl.multiple_of(qb * QBLK, QBLK)
            k0 = pl.multiple_of(jnp.clip(q0 - WIN, 0, SQ - KBLK), WIN)
            return qb, q0, k0

        def issue_kv(j):
            _, _, k0 = window_start(j)
            db = j % 2
            copies = []
            for h in range(HEADS):
                for kv, src in ((0, k_ref), (1, v_ref)):
                    c = pltpu.make_async_copy(
                        src.at[pl.ds(k0, KBLK), my * HEADS + h, :],
                        kv_ref.at[db, kv, h],
                        kv_sems.at[db, kv, h],
                    )
                    c.start()
                    copies.append(c)
            return copies

        def compute_chunk(j, copies):
            qb, q0, k0 = window_start(j)
            db = j % 2
            nxt = issue_kv(j + 1) if j + 1 < N_DEV else []
            q_blk = jnp.dot(
                x_ref[pl.ds(q0, QBLK), :].astype(jnp.bfloat16), wqb_ref[:, :],
                preferred_element_type=jnp.float32,
            ).astype(jnp.bfloat16)
            for h in range(HEADS):
                copies[2 * h].wait()
                copies[2 * h + 1].wait()
                kwin = kv_ref[db, 0, h].astype(jnp.bfloat16)
                vwin = kv_ref[db, 1, h].astype(jnp.bfloat16)
                s = lax.dot_general(
                    q_blk[:, h * DH:(h + 1) * DH], kwin,
                    (((1,), (1,)), ((), ())),
                    preferred_element_type=jnp.float32,
                )
                rows = q0 + lax.broadcasted_iota(jnp.int32, (QBLK, KBLK), 0)
                cols = k0 + lax.broadcasted_iota(jnp.int32, (QBLK, KBLK), 1)
                w = jnp.exp(jnp.where(jnp.abs(rows - cols) <= WIN, s, -1e9))
                p = (w / jnp.sum(w, axis=1, keepdims=True)).astype(jnp.bfloat16)
                ctx = jnp.dot(p, vwin, preferred_element_type=jnp.float32)
                ctx_ref[:, h * DH:(h + 1) * DH] = ctx.astype(jnp.bfloat16)
            partial_ref[pl.ds(q0, QBLK), :] = jnp.dot(
                ctx_ref[:, :], wob_ref[:, :], preferred_element_type=jnp.float32
            ).astype(jnp.bfloat16)
            return nxt

        copies = compute_chunk(0, issue_kv(0))
        for s_hop in range(N_DEV - 1):
            if s_hop == 0:
                src = partial_ref.at[pl.ds(my * CHUNK, CHUNK), :]
            else:
                src = comm_ref.at[s_hop - 1]
            rdma = pltpu.make_async_remote_copy(
                src_ref=src,
                dst_ref=comm_ref.at[s_hop],
                send_sem=send_sems.at[s_hop],
                recv_sem=recv_sems.at[s_hop],
                device_id=(right,),
                device_id_type=pl.DeviceIdType.MESH,
            )
            rdma.start()
            add_chunk = (my - 1 - s_hop) % N_DEV
            copies = compute_chunk(s_hop + 1, copies)
            rdma.wait()
            comm_ref[s_hop] = comm_ref[s_hop] + partial_ref[
                pl.ds(add_chunk * CHUNK, CHUNK), :
            ]

        rc = (my + 1) % N_DEV
        ag_ref[rc] = comm_ref[N_DEV - 2]

        def ag_hop(chunk_id, sem_slot, target):
            slot = chunk_id % N_DEV
            return pltpu.make_async_remote_copy(
                src_ref=ag_ref.at[slot],
                dst_ref=ag_ref.at[slot],
                send_sem=send_sems.at[sem_slot],
                recv_sem=recv_sems.at[sem_slot],
                device_id=(target,),
                device_id_type=pl.DeviceIdType.MESH,
            )

        def to_out(chunk_id):
            slot = chunk_id % N_DEV
            out_ref[pl.ds(slot * CHUNK, CHUNK), :] = ag_ref[slot].astype(
                jnp.float32
            )

        partner = (my + 4) % N_DEV

        cw = ag_hop(my + 1, 7, right)
        ccw = ag_hop(my + 1, 9, left)
        z = ag_hop(my + 1, 11, partner)
        cw.start()
        ccw.start()
        z.start()
        to_out(rc)
        cw.wait()
        ccw.wait()
        z.wait()
        cw = ag_hop(my, 8, right)
        ccw = ag_hop(my + 2, 10, left)
        z = ag_hop(my, 12, partner)
        cw.start()
        ccw.start()
        z.start()
        for c in (my, my + 2, my + 5):
            to_out(c)
        cw.wait()
        ccw.wait()
        z.wait()
        z = ag_hop(my + 2, 13, partner)
        z.start()
        for c in (my - 1, my + 3, my + 4):
            to_out(c)
        z.wait()
        to_out(my + 6)

    out2d = pl.pallas_call(
        body,
        out_shape=jax.ShapeDtypeStruct((SQ, D_MODEL), jnp.float32),
        in_specs=[
            pl.BlockSpec(memory_space=pltpu.VMEM),
            pl.BlockSpec(memory_space=pltpu.VMEM),
            pl.BlockSpec(memory_space=pl.ANY),
            pl.BlockSpec(memory_space=pl.ANY),
            pl.BlockSpec(memory_space=pltpu.VMEM),
        ],
        out_specs=pl.BlockSpec(memory_space=pltpu.VMEM),
        scratch_shapes=[
            pltpu.VMEM((QBLK, D_MODEL), jnp.bfloat16),
            pltpu.VMEM((SQ, D_MODEL), jnp.bfloat16),
            pltpu.VMEM((N_DEV - 1, CHUNK, D_MODEL), jnp.bfloat16),
            pltpu.VMEM((N_DEV, CHUNK, D_MODEL), jnp.bfloat16),
            pltpu.VMEM((D_MODEL, D_MODEL), jnp.bfloat16),
            pltpu.VMEM((D_MODEL, D_MODEL), jnp.bfloat16),
            pltpu.VMEM((2, 2, HEADS, KBLK, DH), jnp.float32),
            pltpu.SemaphoreType.DMA((2 * (N_DEV - 1),)),
            pltpu.SemaphoreType.DMA((2 * (N_DEV - 1),)),
            pltpu.SemaphoreType.DMA((2, 2, HEADS)),
        ],
        compiler_params=pltpu.CompilerParams(
            collective_id=0, vmem_limit_bytes=96 * 1024 * 1024
        ),
    )(xb, Wq, kb, vb, Wo)
    return out2d.reshape(1, SQ, D_MODEL)


# device time: 108808 ns/iter; 1.0345x vs baseline; 1.0345x over previous
import functools

import jax
import jax.numpy as jnp
from jax import lax
from jax.experimental import pallas as pl
from jax.experimental.pallas import tpu as pltpu

N_DEV = 8
SQ = 2048
D_MODEL = 1024
HEADS = 8
DH = 128
WIN = 128
QBLK = 256
KBLK = 512
CHUNK = SQ // N_DEV
SCALE = 0.08838834764831843


def kernel(x, Wq, K_ext, V_ext, Wo):
    xb = x[0]
    kb = K_ext[0]
    vb = V_ext[0]

    def body(x_ref, wq_ref, k_ref, v_ref, wo_ref, out_ref,
             ctx_ref, partial_ref, comm_ref, ag_ref,
             wqb_ref, wob_ref, kv_ref, send_sems, recv_sems, kv_sems):
        my = lax.axis_index("i")
        left = (my - 1) % N_DEV
        right = (my + 1) % N_DEV

        barrier_sem = pltpu.get_barrier_semaphore()
        for nbr in (left, right):
            pl.semaphore_signal(
                barrier_sem, inc=1,
                device_id=(nbr,), device_id_type=pl.DeviceIdType.MESH,
            )
        wqb_ref[:, :] = (wq_ref[:, :] * SCALE).astype(jnp.bfloat16)
        wob_ref[:, :] = wo_ref[:, :].astype(jnp.bfloat16)
        pl.semaphore_wait(barrier_sem, 2)

        def window_start(j):
            qb = (my - j) % N_DEV
            q0 = pl.multiple_of(qb * QBLK, QBLK)
            k0 = pl.multiple_of(jnp.clip(q0 - WIN, 0, SQ - KBLK), WIN)
            return qb, q0, k0

        def issue_kv(j):
            _, _, k0 = window_start(j)
            db = j % 2
            copies = []
            for h in range(HEADS):
                for kv, src in ((0, k_ref), (1, v_ref)):
                    c = pltpu.make_async_copy(
                        src.at[pl.ds(k0, KBLK), my * HEADS + h, :],
                        kv_ref.at[db, kv, h],
                        kv_sems.at[db, kv, h],
                    )
                    c.start()
                    copies.append(c)
            return copies

        def compute_chunk(j, copies):
            qb, q0, k0 = window_start(j)
            db = j % 2
            nxt = issue_kv(j + 1) if j + 1 < N_DEV else []
            q_blk = jnp.dot(
                x_ref[pl.ds(q0, QBLK), :].astype(jnp.bfloat16), wqb_ref[:, :],
                preferred_element_type=jnp.float32,
            ).astype(jnp.bfloat16)
            rows = q0 + lax.broadcasted_iota(jnp.int32, (QBLK, KBLK), 0)
            cols = k0 + lax.broadcasted_iota(jnp.int32, (QBLK, KBLK), 1)
            bias = jnp.where(jnp.abs(rows - cols) <= WIN, 0.0, -1e9)
            for h in range(HEADS):
                copies[2 * h].wait()
                copies[2 * h + 1].wait()
                kwin = kv_ref[db, 0, h].astype(jnp.bfloat16)
                vwin = kv_ref[db, 1, h].astype(jnp.bfloat16)
                s = lax.dot_general(
                    q_blk[:, h * DH:(h + 1) * DH], kwin,
                    (((1,), (1,)), ((), ())),
                    preferred_element_type=jnp.float32,
                )
                w = jnp.exp(s + bias)
                inv = 1.0 / jnp.sum(w, axis=1, keepdims=True)
                ctx = jnp.dot(
                    w.astype(jnp.bfloat16), vwin,
                    preferred_element_type=jnp.float32,
                )
                ctx_ref[:, h * DH:(h + 1) * DH] = (ctx * inv).astype(
                    jnp.bfloat16
                )
            partial_ref[pl.ds(q0, QBLK), :] = jnp.dot(
                ctx_ref[:, :], wob_ref[:, :], preferred_element_type=jnp.float32
            ).astype(jnp.bfloat16)
            return nxt

        copies = compute_chunk(0, issue_kv(0))
        for s_hop in range(N_DEV - 1):
            if s_hop == 0:
                src = partial_ref.at[pl.ds(my * CHUNK, CHUNK), :]
            else:
                src = comm_ref.at[s_hop - 1]
            rdma = pltpu.make_async_remote_copy(
                src_ref=src,
                dst_ref=comm_ref.at[s_hop],
                send_sem=send_sems.at[s_hop],
                recv_sem=recv_sems.at[s_hop],
                device_id=(right,),
                device_id_type=pl.DeviceIdType.MESH,
            )
            rdma.start()
            add_chunk = (my - 1 - s_hop) % N_DEV
            copies = compute_chunk(s_hop + 1, copies)
            rdma.wait()
            comm_ref[s_hop] = comm_ref[s_hop] + partial_ref[
                pl.ds(add_chunk * CHUNK, CHUNK), :
            ]

        rc = (my + 1) % N_DEV
        ag_ref[rc] = comm_ref[N_DEV - 2]

        def ag_hop(chunk_id, sem_slot, target):
            slot = chunk_id % N_DEV
            return pltpu.make_async_remote_copy(
                src_ref=ag_ref.at[slot],
                dst_ref=ag_ref.at[slot],
                send_sem=send_sems.at[sem_slot],
                recv_sem=recv_sems.at[sem_slot],
                device_id=(target,),
                device_id_type=pl.DeviceIdType.MESH,
            )

        def to_out(chunk_id):
            slot = chunk_id % N_DEV
            out_ref[pl.ds(slot * CHUNK, CHUNK), :] = ag_ref[slot].astype(
                jnp.float32
            )

        pending = [rc]
        for t in range(4):
            cw = ag_hop(my + 1 - t, N_DEV - 1 + t, right)
            cw.start()
            if t < 3:
                ccw = ag_hop(my + 1 + t, N_DEV + 3 + t, left)
                ccw.start()
            for c in pending:
                to_out(c)
            pending = []
            cw.wait()
            pending.append(my - t)
            if t < 3:
                ccw.wait()
                pending.append(my + 2 + t)
        for c in pending:
            to_out(c)

    out2d = pl.pallas_call(
        body,
        out_shape=jax.ShapeDtypeStruct((SQ, D_MODEL), jnp.float32),
        in_specs=[
            pl.BlockSpec(memory_space=pltpu.VMEM),
            pl.BlockSpec(memory_space=pltpu.VMEM),
            pl.BlockSpec(memory_space=pl.ANY),
            pl.BlockSpec(memory_space=pl.ANY),
            pl.BlockSpec(memory_space=pltpu.VMEM),
        ],
        out_specs=pl.BlockSpec(memory_space=pltpu.VMEM),
        scratch_shapes=[
            pltpu.VMEM((QBLK, D_MODEL), jnp.bfloat16),
            pltpu.VMEM((SQ, D_MODEL), jnp.bfloat16),
            pltpu.VMEM((N_DEV - 1, CHUNK, D_MODEL), jnp.bfloat16),
            pltpu.VMEM((N_DEV, CHUNK, D_MODEL), jnp.bfloat16),
            pltpu.VMEM((D_MODEL, D_MODEL), jnp.bfloat16),
            pltpu.VMEM((D_MODEL, D_MODEL), jnp.bfloat16),
            pltpu.VMEM((2, 2, HEADS, KBLK, DH), jnp.float32),
            pltpu.SemaphoreType.DMA((2 * (N_DEV - 1),)),
            pltpu.SemaphoreType.DMA((2 * (N_DEV - 1),)),
            pltpu.SemaphoreType.DMA((2, 2, HEADS)),
        ],
        compiler_params=pltpu.CompilerParams(
            collective_id=0, vmem_limit_bytes=96 * 1024 * 1024
        ),
    )(xb, Wq, kb, vb, Wo)
    return out2d.reshape(1, SQ, D_MODEL)
